# baseline (speedup 1.0000x reference)
import jax
import jax.numpy as jnp
from jax import lax
from jax.experimental import pallas as pl
from jax.experimental.pallas import tpu as pltpu
from jax.experimental.pallas import tpu_sc as plsc

B = 4096
S = 200
H = 64
NC = 2
NS = 16
NW = NC * NS
LANES = 128
RING = 5


def _body(x4_hbm, emb_hbm, pos_hbm, out5_hbm, idx_t, pos_v, *bufs):
    rows = bufs[0:RING]
    chunk = bufs[RING:2 * RING]
    gat_sems = bufs[2 * RING:3 * RING]
    out_sems = bufs[3 * RING:4 * RING]
    wid = lax.axis_index("s") * NC + lax.axis_index("c")

    pltpu.sync_copy(x4_hbm.at[:, wid], idx_t)
    pltpu.sync_copy(pos_hbm, pos_v)

    iota16 = lax.iota(jnp.int32, 16)

    def gather_desc(p, b):
        return pltpu.make_async_copy(
            emb_hbm.at[idx_t.at[p // 8, p % 8]], rows[b], gat_sems[b])

    def out_desc(p, b):
        return pltpu.make_async_copy(
            chunk[b].at[:, :, pl.ds(0, LANES)], out5_hbm.at[p, :, wid],
            out_sems[b])

    hh_c = [(iota16 + 16 * j) // 8 for j in range(H // 16)]
    hl_c = [(iota16 + 16 * j) % 8 for j in range(H // 16)]

    def transpose_add(p, b):
        pvs = [pos_v[p, pl.ds(16 * j, 16)] for j in range(H // 16)]

        @plsc.parallel_loop(0, LANES, 1, unroll=8)
        def _(m):
            mvec = jnp.broadcast_to(m, (16,))
            for j in range(H // 16):
                v = rows[b][m, pl.ds(16 * j, 16)] + pvs[j]
                plsc.store_scatter(chunk[b], [hh_c[j], hl_c[j], mvec], v)

    def step(p, b, do_wait_out=True, do_fire=True):
        if do_fire:
            gather_desc(p + RING - 1, (b + RING - 1) % RING).start()
        gather_desc(p, b).wait()
        if do_wait_out:
            out_desc(p - RING, b).wait()
        transpose_add(p, b)
        out_desc(p, b).start()

    for p in range(RING - 1):
        gather_desc(p, p).start()
    for p in range(RING):
        step(p, p, do_wait_out=False)

    def steady(g, carry):
        p0 = RING + g * RING
        for b in range(RING):
            step(p0 + b, b)
        return carry

    lax.fori_loop(0, (S - 2 * RING) // RING, steady, 0)

    for p in range(S - RING, S):
        step(p, p % RING, do_fire=(p + RING - 1 < S))
    for p in range(S - RING, S):
        out_desc(p, p % RING).wait()


@jax.jit
def _embed(x4, emb_table, pos_table):
    mesh = plsc.VectorSubcoreMesh(core_axis_name="c", subcore_axis_name="s",
                                  num_cores=NC, num_subcores=NS)
    run = pl.kernel(
        _body,
        out_type=jax.ShapeDtypeStruct((S, 8, NW, 8, LANES), jnp.float32),
        mesh=mesh,
        scratch_types=(
            [pltpu.VMEM((25, 8, LANES), jnp.int32),
             pltpu.VMEM((S, H), jnp.float32)]
            + [pltpu.VMEM((LANES, H), jnp.float32)] * RING
            + [pltpu.VMEM((8, 8, LANES + 1), jnp.float32)] * RING
            + [pltpu.SemaphoreType.DMA] * (2 * RING)
        ),
        compiler_params=pltpu.CompilerParams(use_tc_tiling_on_sc=False,
                                             needs_layout_passes=False),
    )
    return run(x4, emb_table, pos_table)


def kernel(x, emb_table, pos_table):
    x4 = x.astype(jnp.int32).T.reshape(25, 8, NW, LANES).transpose((0, 2, 1, 3))
    out5 = _embed(x4, emb_table, pos_table)
    return out5.transpose((2, 4, 0, 1, 3)).reshape(B, S, H)


# speedup vs baseline: 1.1190x; 1.1190x over previous
import jax
import jax.numpy as jnp
from jax import lax
from jax.experimental import pallas as pl
from jax.experimental.pallas import tpu as pltpu
from jax.experimental.pallas import tpu_sc as plsc

B = 4096
S = 200
H = 64
NC = 2
NS = 16
NW = NC * NS
LANES = 128
RING = 4


def _body(x4_hbm, emb_hbm, pos_hbm, out5_hbm, idx_t, pos_v, *bufs):
    rows = bufs[0:RING]
    chunk = bufs[RING:2 * RING]
    gat_sems = bufs[2 * RING:3 * RING]
    out_sems = bufs[3 * RING:4 * RING]
    wid = lax.axis_index("s") * NC + lax.axis_index("c")

    pltpu.sync_copy(x4_hbm.at[:, wid], idx_t)
    pltpu.sync_copy(pos_hbm, pos_v)

    iota16 = lax.iota(jnp.int32, 16)

    def gather_desc(p, b):
        return pltpu.make_async_copy(
            emb_hbm.at[idx_t.at[p // 8, p % 8]], rows[b], gat_sems[b])

    def out_desc(p, b):
        return pltpu.make_async_copy(
            chunk[b].at[:, :, pl.ds(0, LANES)], out5_hbm.at[p, :, wid],
            out_sems[b])

    hh_c = [(iota16 + 16 * j) // 8 for j in range(H // 16)]
    hl_c = [(iota16 + 16 * j) % 8 for j in range(H // 16)]

    def transpose_add(p, b):
        pvs = [pos_v[p, pl.ds(16 * j, 16)] for j in range(H // 16)]

        @plsc.parallel_loop(0, LANES, 1, unroll=4)
        def _(m):
            mvec = jnp.broadcast_to(m, (16,))
            for j in range(H // 16):
                v = rows[b][m, pl.ds(16 * j, 16)] + pvs[j]
                plsc.store_scatter(chunk[b], [hh_c[j], hl_c[j], mvec], v)

    def step(p, b, do_wait_out=True, do_fire=True):
        if do_fire:
            gather_desc(p + 3, (b + 3) % RING).start()
        gather_desc(p, b).wait()
        if do_wait_out:
            out_desc(p - RING, b).wait()
        transpose_add(p, b)
        out_desc(p, b).start()

    gather_desc(0, 0).start()
    gather_desc(1, 1).start()
    gather_desc(2, 2).start()
    for p in range(RING):
        step(p, p, do_wait_out=False)

    def steady(g, carry):
        p0 = RING + g * RING
        for b in range(RING):
            step(p0 + b, b)
        return carry

    lax.fori_loop(0, (S - 2 * RING) // RING, steady, 0)

    step(S - 4, 0)
    step(S - 3, 1, do_fire=False)
    step(S - 2, 2, do_fire=False)
    step(S - 1, 3, do_fire=False)
    for p in range(S - RING, S):
        out_desc(p, p % RING).wait()


@jax.jit
def _embed(x4, emb_table, pos_table):
    mesh = plsc.VectorSubcoreMesh(core_axis_name="c", subcore_axis_name="s",
                                  num_cores=NC, num_subcores=NS)
    run = pl.kernel(
        _body,
        out_type=jax.ShapeDtypeStruct((S, 8, NW, 8, LANES), jnp.float32),
        mesh=mesh,
        scratch_types=(
            [pltpu.VMEM((25, 8, LANES), jnp.int32),
             pltpu.VMEM((S, H), jnp.float32)]
            + [pltpu.VMEM((LANES, H), jnp.float32)] * RING
            + [pltpu.VMEM((8, 8, LANES + 1), jnp.float32)] * RING
            + [pltpu.SemaphoreType.DMA] * (2 * RING)
        ),
        compiler_params=pltpu.CompilerParams(use_tc_tiling_on_sc=False,
                                             needs_layout_passes=False),
    )
    return run(x4, emb_table, pos_table)


def kernel(x, emb_table, pos_table):
    x4 = x.astype(jnp.int32).T.reshape(25, 8, NW, LANES).transpose((0, 2, 1, 3))
    out5 = _embed(x4, emb_table, pos_table)
    return out5.transpose((2, 4, 0, 1, 3)).reshape(B, S, H)


# speedup vs baseline: 1.1221x; 1.0028x over previous
import jax
import jax.numpy as jnp
from jax import lax
from jax.experimental import pallas as pl
from jax.experimental.pallas import tpu as pltpu
from jax.experimental.pallas import tpu_sc as plsc

B = 4096
S = 200
H = 64
NC = 2
NS = 16
NW = NC * NS
LANES = 128
RING = 4


def _body(x4_hbm, emb_hbm, pos_hbm, out5_hbm, idx_t, pos_v, *bufs):
    rows = bufs[0:RING]
    chunk = bufs[RING:2 * RING]
    gat_sems = bufs[2 * RING:3 * RING]
    out_sems = bufs[3 * RING:4 * RING]
    wid = lax.axis_index("s") * NC + lax.axis_index("c")

    pltpu.sync_copy(x4_hbm.at[:, wid], idx_t)
    pltpu.sync_copy(pos_hbm, pos_v)

    iota16 = lax.iota(jnp.int32, 16)

    def gather_descs(p, b):
        return (
            pltpu.make_async_copy(
                emb_hbm.at[idx_t.at[p // 8, p % 8, pl.ds(0, 64)]],
                rows[b].at[pl.ds(0, 64)], gat_sems[b]),
            pltpu.make_async_copy(
                emb_hbm.at[idx_t.at[p // 8, p % 8, pl.ds(64, 64)]],
                rows[b].at[pl.ds(64, 64)], gat_sems[b]),
        )

    def out_desc(p, b):
        return pltpu.make_async_copy(
            chunk[b].at[:, :, pl.ds(0, LANES)], out5_hbm.at[p, :, wid],
            out_sems[b])

    hh_c = [(iota16 + 16 * j) // 8 for j in range(H // 16)]
    hl_c = [(iota16 + 16 * j) % 8 for j in range(H // 16)]

    def transpose_add(p, b):
        pvs = [pos_v[p, pl.ds(16 * j, 16)] for j in range(H // 16)]

        @plsc.parallel_loop(0, LANES, 1, unroll=4)
        def _(m):
            mvec = jnp.broadcast_to(m, (16,))
            for j in range(H // 16):
                v = rows[b][m, pl.ds(16 * j, 16)] + pvs[j]
                plsc.store_scatter(chunk[b], [hh_c[j], hl_c[j], mvec], v)

    def step(p, b, do_wait_out=True, do_fire=True):
        if do_fire:
            for d in gather_descs(p + 3, (b + 3) % RING):
                d.start()
        for d in gather_descs(p, b):
            d.wait()
        if do_wait_out:
            out_desc(p - RING, b).wait()
        transpose_add(p, b)
        out_desc(p, b).start()

    for p0 in range(3):
        for d in gather_descs(p0, p0):
            d.start()
    for p in range(RING):
        step(p, p, do_wait_out=False)

    def steady(g, carry):
        p0 = RING + g * RING
        for b in range(RING):
            step(p0 + b, b)
        return carry

    lax.fori_loop(0, (S - 2 * RING) // RING, steady, 0)

    step(S - 4, 0)
    step(S - 3, 1, do_fire=False)
    step(S - 2, 2, do_fire=False)
    step(S - 1, 3, do_fire=False)
    for p in range(S - RING, S):
        out_desc(p, p % RING).wait()


@jax.jit
def _embed(x4, emb_table, pos_table):
    mesh = plsc.VectorSubcoreMesh(core_axis_name="c", subcore_axis_name="s",
                                  num_cores=NC, num_subcores=NS)
    run = pl.kernel(
        _body,
        out_type=jax.ShapeDtypeStruct((S, 8, NW, 8, LANES), jnp.float32),
        mesh=mesh,
        scratch_types=(
            [pltpu.VMEM((25, 8, LANES), jnp.int32),
             pltpu.VMEM((S, H), jnp.float32)]
            + [pltpu.VMEM((LANES, H), jnp.float32)] * RING
            + [pltpu.VMEM((8, 8, LANES + 1), jnp.float32)] * RING
            + [pltpu.SemaphoreType.DMA] * (2 * RING)
        ),
        compiler_params=pltpu.CompilerParams(use_tc_tiling_on_sc=False,
                                             needs_layout_passes=False),
    )
    return run(x4, emb_table, pos_table)


def kernel(x, emb_table, pos_table):
    x4 = x.astype(jnp.int32).T.reshape(25, 8, NW, LANES).transpose((0, 2, 1, 3))
    out5 = _embed(x4, emb_table, pos_table)
    return out5.transpose((2, 4, 0, 1, 3)).reshape(B, S, H)
